# baseline (speedup 1.0000x reference)
import functools

import jax
import jax.numpy as jnp
from jax import lax
from jax.experimental import pallas as pl
from jax.experimental.pallas import tpu as pltpu
from jax.experimental.pallas import tpu_sc as plsc

B, Cin, C, H, W = 2, 64, 64, 224, 224
C2 = 2 * C
Ns, K = 12544, 9
HW = H * W
HID = (2 * C + 3) // 2
HWp = HW + 16
NW = 32
NKN = B * K * Ns
NN_PER_W = NKN // NW
GCH = 392
PT_PER_W = (B * Ns) // NW
PT_PER_T = Ns // 16

_MESH = plsc.VectorSubcoreMesh(core_axis_name="c", subcore_axis_name="s")


RB = 1792
NB = HW // RB


def _conv1_body(xt_ref, xm_ref, xb_ref, wtab_ref, wf1_ref, b0_ref, b1_ref,
                tab_ref, f1_ref, cat_ref):
    i = pl.program_id(1)
    zero = jnp.zeros((RB, C2), jnp.bfloat16)
    cat_ref[pl.ds(0, RB), :] = jnp.where(i == 0, zero, xt_ref[0])
    cat_ref[pl.ds(RB, RB), :] = xm_ref[0]
    cat_ref[pl.ds(2 * RB, RB), :] = jnp.where(i == NB - 1, zero, xb_ref[0])
    wiota = lax.broadcasted_iota(jnp.int32, (RB, C2), 0) % W
    m_l = wiota != 0
    m_r = wiota != (W - 1)
    acc0 = jnp.zeros((RB, C2), jnp.float32)
    acc1 = jnp.zeros((RB, C2), jnp.float32)
    for t in range(9):
        dy, dx = t // 3 - 1, t % 3 - 1
        xs = cat_ref[pl.ds(RB + dy * W + dx, RB), :]
        if dx == -1:
            xs = jnp.where(m_l, xs, jnp.bfloat16(0))
        elif dx == 1:
            xs = jnp.where(m_r, xs, jnp.bfloat16(0))
        acc0 = acc0 + jnp.dot(xs, wtab_ref[t], preferred_element_type=jnp.float32)
        acc1 = acc1 + jnp.dot(xs, wf1_ref[t], preferred_element_type=jnp.float32)
    tab_ref[0] = jnp.maximum(acc0 + b0_ref[...], 0.0)
    f1_ref[0] = acc1 + b1_ref[...]


def _conv1(x_cat, wtab, wf1, b0p, b1p):
    blk = pl.BlockSpec((1, RB, C2), lambda b, i: (b, i, 0))
    full3 = pl.BlockSpec(None, lambda b, i: (0, 0, 0))
    full2 = pl.BlockSpec(None, lambda b, i: (0, 0))
    return pl.pallas_call(
        _conv1_body,
        grid=(B, NB),
        in_specs=[
            pl.BlockSpec((1, RB, C2), lambda b, i: (b, jnp.maximum(i - 1, 0), 0)),
            blk,
            pl.BlockSpec((1, RB, C2),
                         lambda b, i: (b, jnp.minimum(i + 1, NB - 1), 0)),
            full3, full3, full2, full2,
        ],
        out_specs=[
            pl.BlockSpec((1, RB, C2), lambda b, i: (b, i, 0)),
            pl.BlockSpec((1, RB, C2), lambda b, i: (b, i, 0)),
        ],
        out_shape=[
            jax.ShapeDtypeStruct((B, HWp, C2), jnp.float32),
            jax.ShapeDtypeStruct((B, HW, C2), jnp.float32),
        ],
        scratch_shapes=[pltpu.VMEM((3 * RB, C2), jnp.bfloat16)],
    )(x_cat, x_cat, x_cat, wtab, wf1, b0p, b1p)


def _conv2_body(tt_ref, tm_ref, tb_ref, f1_ref, w2_ref, b2_ref,
                out_ref, cat_ref):
    i = pl.program_id(1)
    zero = jnp.zeros((RB, C2), jnp.bfloat16)
    cat_ref[pl.ds(0, RB), :] = jnp.where(i == 0, zero,
                                         tt_ref[0].astype(jnp.bfloat16))
    cat_ref[pl.ds(RB, RB), :] = tm_ref[0].astype(jnp.bfloat16)
    cat_ref[pl.ds(2 * RB, RB), :] = jnp.where(i == NB - 1, zero,
                                              tb_ref[0].astype(jnp.bfloat16))
    wiota = lax.broadcasted_iota(jnp.int32, (RB, C2), 0) % W
    m_l = wiota != 0
    m_r = wiota != (W - 1)
    acc = jnp.zeros((RB, C2), jnp.float32)
    for t in range(9):
        dy, dx = t // 3 - 1, t % 3 - 1
        xs = cat_ref[pl.ds(RB + dy * W + dx, RB), :]
        if dx == -1:
            xs = jnp.where(m_l, xs, 0.0)
        elif dx == 1:
            xs = jnp.where(m_r, xs, 0.0)
        acc = acc + jnp.dot(xs, w2_ref[t], preferred_element_type=jnp.float32)
    out_ref[0] = jnp.maximum(acc + b2_ref[...] + f1_ref[0], 0.0)


def _conv2(tab2, f1, w2p, b2p):
    full3 = pl.BlockSpec(None, lambda b, i: (0, 0, 0))
    full2 = pl.BlockSpec(None, lambda b, i: (0, 0))
    return pl.pallas_call(
        _conv2_body,
        grid=(B, NB),
        in_specs=[
            pl.BlockSpec((1, RB, C2), lambda b, i: (b, jnp.maximum(i - 1, 0), 0)),
            pl.BlockSpec((1, RB, C2), lambda b, i: (b, i, 0)),
            pl.BlockSpec((1, RB, C2),
                         lambda b, i: (b, jnp.minimum(i + 1, NB - 1), 0)),
            pl.BlockSpec((1, RB, C2), lambda b, i: (b, i, 0)),
            full3, full2,
        ],
        out_specs=pl.BlockSpec((1, RB, C2), lambda b, i: (b, i, 0)),
        out_shape=jax.ShapeDtypeStruct((B, HW, C2), jnp.float32),
        scratch_shapes=[pltpu.VMEM((3 * RB, C2), jnp.bfloat16)],
    )(tab2, tab2, tab2, f1, w2p, b2p)


@functools.partial(
    pl.kernel,
    out_type=(
        jax.ShapeDtypeStruct((NKN, C2), jnp.float32),
        jax.ShapeDtypeStruct((B * Ns, C2), jnp.float32),
    ),
    mesh=_MESH,
    scratch_types=[
        pltpu.VMEM((GCH,), jnp.int32),
        pltpu.VMEM((GCH,), jnp.int32),
        pltpu.VMEM((GCH, C2), jnp.float32),
        pltpu.VMEM((GCH, C2), jnp.float32),
        pltpu.SemaphoreType.DMA,
        pltpu.SemaphoreType.DMA,
    ],
    compiler_params=pltpu.CompilerParams(needs_layout_passes=False),
)
def _sc_gather(tab, nbrs_g, pc_g, nn_out, pt_out, idx_a, idx_b,
               rows_a, rows_b, gsem, osem):
    wid = lax.axis_index("c") * 16 + lax.axis_index("s")
    nb_base = wid * NN_PER_W
    pt_base = wid * PT_PER_W
    nnc = NN_PER_W // GCH
    ptc = PT_PER_W // GCH

    def src_dst(i):
        if i < nnc:
            return nbrs_g, nn_out, nb_base + i * GCH
        j = i - nnc
        return pc_g, pt_out, pt_base + j * GCH

    n = nnc + ptc
    idx_bufs = (idx_a, idx_b)
    row_bufs = (rows_a, rows_b)
    gathers = {}
    outs = {}
    src0, _, base0 = src_dst(0)
    pltpu.sync_copy(src0.at[pl.ds(base0, GCH)], idx_a)
    gathers[0] = pltpu.async_copy(tab.at[idx_a], rows_a, gsem)
    for i in range(n):
        s, nx = i % 2, (i + 1) % 2
        if i + 1 < n:
            srcn, _, basen = src_dst(i + 1)
            pltpu.sync_copy(srcn.at[pl.ds(basen, GCH)], idx_bufs[nx])
        gathers[i].wait()
        if i >= 1:
            outs[i - 1].wait()
        _, dsti, basei = src_dst(i)
        outs[i] = pltpu.async_copy(row_bufs[s], dsti.at[pl.ds(basei, GCH)],
                                   osem)
        if i + 1 < n:
            gathers[i + 1] = pltpu.async_copy(tab.at[idx_bufs[nx]],
                                              row_bufs[nx], gsem)
    outs[n - 1].wait()


BN = 896


def _attn_body(nn_ref, pt_ref, disp_ref, g_ref, pw_ref, wp_ref, w2_ref,
               b1_ref, b2_ref, bias_ref, out_ref):
    pt = pt_ref[0]
    point = (jnp.dot(pt, pw_ref[...], preferred_element_type=jnp.float32)
             + b1_ref[...])
    scores = []
    for k in range(K):
        h = (jnp.dot(nn_ref[0, k], g_ref[...],
                     preferred_element_type=jnp.float32)
             + lax.dot_general(disp_ref[0, pl.ds(3 * k, 3)], wp_ref[...],
                               (((0,), (0,)), ((), ())),
                               preferred_element_type=jnp.float32)
             + point)
        h = jnp.where(h >= 0, h, 0.2 * h)
        scores.append(jnp.dot(h, w2_ref[...], preferred_element_type=jnp.float32)
                      + b2_ref[...])
    exps = [jnp.exp(s) for s in scores]
    den = exps[0]
    for k in range(1, K):
        den = den + exps[k]
    inv = 1.0 / den
    accd = jnp.zeros((BN, C), jnp.float32)
    accr = jnp.zeros((BN, C), jnp.float32)
    for k in range(K):
        a = exps[k] * inv
        nk = nn_ref[0, k][:, 0:C]
        accd = accd + a[:, 0:1] * nk
        accr = accr + a[:, 1:2] * nk
    acc = jnp.concatenate([accd, accr], axis=1) + bias_ref[...]
    ci = lax.broadcasted_iota(jnp.int32, (BN, C2), 1)
    keep = jnp.logical_and(ci != 0, ci != C)
    out_ref[0] = acc + jnp.where(keep, pt, 0.0)


def _tc_attn(nn, pt, disp, g, pw, wp, w2, b1s, b2s, bias):
    nblk = Ns // BN
    w2d = pl.BlockSpec(None, lambda b, i: (0, 0))
    return pl.pallas_call(
        _attn_body,
        grid=(B, nblk),
        in_specs=[
            pl.BlockSpec((1, K, BN, C2), lambda b, i: (b, 0, i, 0)),
            pl.BlockSpec((1, BN, C2), lambda b, i: (b, i, 0)),
            pl.BlockSpec((1, 3 * K, BN), lambda b, i: (b, 0, i)),
            w2d, w2d, w2d, w2d, w2d, w2d, w2d,
        ],
        out_specs=pl.BlockSpec((1, BN, C2), lambda b, i: (b, i, 0)),
        out_shape=jax.ShapeDtypeStruct((B, Ns, C2), jnp.float32),
    )(nn, pt, disp, g, pw, wp, w2, b1s, b2s, bias)


CPR = HW // 16
CCH = 384
_PASSES = ((0, 384), (384, 384), (768, 16))


@functools.partial(
    pl.kernel,
    out_type=jax.ShapeDtypeStruct((B * HWp, C2), jnp.float32),
    mesh=_MESH,
    scratch_types=[
        pltpu.VMEM((HW,), jnp.int32),
        pltpu.VMEM((Ns,), jnp.int32),
        pltpu.VMEM((PT_PER_T,), jnp.int32),
        pltpu.VMEM((384,), jnp.int32),
        pltpu.VMEM((384, C2), jnp.float32),
        pltpu.SemaphoreType.DMA,
    ],
    compiler_params=pltpu.CompilerParams(needs_layout_passes=False),
)
def _sc_scatter(tab, rows, pc_lf, pc_g, out,
                wm_v, pc_v, msk_v, idx_v, rows_v, sem):
    b = lax.axis_index("c")
    t = lax.axis_index("s")

    row0 = b * HWp + t * CPR

    def cp_chunk(j, _):
        pltpu.sync_copy(tab.at[pl.ds(row0 + j * CCH, CCH)], rows_v)
        pltpu.sync_copy(rows_v, out.at[pl.ds(row0 + j * CCH, CCH)])
        return _

    lax.fori_loop(0, CPR // CCH, cp_chunk, 0)
    tail = CPR - (CPR // CCH) * CCH
    pltpu.sync_copy(tab.at[pl.ds(row0 + CPR - tail, tail)],
                    rows_v.at[pl.ds(0, tail)])
    pltpu.sync_copy(rows_v.at[pl.ds(0, tail)],
                    out.at[pl.ds(row0 + CPR - tail, tail)])

    pltpu.sync_copy(pc_lf.at[pl.ds(b * Ns, Ns)], pc_v)
    lanes = lax.iota(jnp.int32, 16)

    def scat(i, _):
        idx16 = pc_v[pl.ds(i * 16, 16)]
        plsc.store_scatter(wm_v, [idx16], lanes + i * 16)
        return _

    lax.fori_loop(0, Ns // 16, scat, 0)

    tb = t * PT_PER_T

    def wmask(i, _):
        idx16 = pc_v[pl.ds(tb + i * 16, 16)]
        got = plsc.load_gather(wm_v, [idx16])
        msk_v[pl.ds(i * 16, 16)] = jnp.where(got == lanes + (tb + i * 16), 1, 0)
        return _

    lax.fori_loop(0, PT_PER_T // 16, wmask, 0)

    plsc.subcore_barrier()

    sentinel = b * HWp + HW + t
    gstart = b * Ns + tb

    for off, npt in _PASSES:
        pltpu.sync_copy(pc_g.at[pl.ds(gstart + off, npt)],
                        idx_v.at[pl.ds(0, npt)])
        pltpu.sync_copy(rows.at[pl.ds(gstart + off, npt)],
                        rows_v.at[pl.ds(0, npt)])

        def effidx(i, _):
            w16 = msk_v[pl.ds(off + i * 16, 16)]
            i16 = idx_v[pl.ds(i * 16, 16)]
            idx_v[pl.ds(i * 16, 16)] = jnp.where(w16 == 1, i16, sentinel)
            return _

        lax.fori_loop(0, npt // 16, effidx, 0)

        pltpu.async_copy(rows_v.at[pl.ds(0, npt)],
                         out.at[idx_v.at[pl.ds(0, npt)]], sem).wait()


def _taps(w):
    return w.transpose(2, 3, 1, 0).reshape(9, Cin, C)


def _packtaps(wd, wr):
    z = jnp.zeros((9, C2, C2), jnp.float32)
    return z.at[:, :C, :C].set(_taps(wd)).at[:, C:, C:].set(_taps(wr))


def kernel(rgb, sdepth, pc_idx, nbrs_idx, nbrs_disp,
           d_w0, d_b0, d_w1, d_b1, d_w2, d_b2,
           r_w0, r_b0, r_w1, r_b1, r_w2, r_b2,
           d_mlp_w1, d_mlp_b1, d_mlp_w2, d_mlp_b2,
           r_mlp_w1, r_mlp_b1, r_mlp_w2, r_mlp_b2,
           d_bias, r_bias):
    x_cat = jnp.concatenate([sdepth, rgb], axis=1).transpose(0, 2, 3, 1)
    x_cat = x_cat.reshape(B, HW, C2).astype(jnp.bfloat16)

    tab3, f1 = _conv1(x_cat,
                      _packtaps(d_w0, r_w0).astype(jnp.bfloat16),
                      _packtaps(d_w1, r_w1).astype(jnp.bfloat16),
                      jnp.concatenate([d_b0, r_b0]).reshape(1, C2),
                      jnp.concatenate([d_b1, r_b1]).reshape(1, C2))
    tab = tab3.reshape(B * HWp, C2)

    off_b = jnp.arange(B, dtype=jnp.int32) * HWp
    pc_l = pc_idx.reshape(B, Ns).astype(jnp.int32)
    pc_g = (pc_l + off_b[:, None]).reshape(-1)
    nbrs = nbrs_idx.reshape(B, Ns, K).astype(jnp.int32).transpose(0, 2, 1)
    nbrs_g = (nbrs + off_b[:, None, None]).reshape(-1)
    disp_t = nbrs_disp.transpose(0, 3, 1, 2).reshape(B, 3 * K, Ns)

    nn_f, pt_f = _sc_gather(tab, nbrs_g, pc_g)

    a1d, a2d, a3d = (d_mlp_w1[:, :C], d_mlp_w1[:, C:2 * C], d_mlp_w1[:, 2 * C:])
    a1r, a2r, a3r = (r_mlp_w1[:, :C], r_mlp_w1[:, C:2 * C], r_mlp_w1[:, 2 * C:])
    g_w = jnp.concatenate([(a1d + a2d).T, (a1r + a2r).T], axis=1)
    g_w = jnp.pad(g_w, ((0, C), (0, 0)))
    pw = jnp.concatenate(
        [jnp.concatenate([-a1d.T, -a1r.T], axis=1),
         jnp.concatenate([-a2d.T, -a2r.T], axis=1)], axis=0)
    wp = jnp.concatenate([a3d.T, a3r.T], axis=1)
    w2 = jnp.zeros((2 * HID, 2), jnp.float32)
    w2 = w2.at[:HID, 0].set(d_mlp_w2[0]).at[HID:, 1].set(r_mlp_w2[0])
    b1s = jnp.concatenate([d_mlp_b1, r_mlp_b1]).reshape(1, 2 * HID)
    b2s = jnp.concatenate([d_mlp_b2, r_mlp_b2]).reshape(1, 2)
    bias = jnp.concatenate([d_bias, r_bias]).reshape(1, C2)

    new_rows = _tc_attn(nn_f.reshape(B, K, Ns, C2), pt_f.reshape(B, Ns, C2),
                        disp_t, g_w, pw, wp, w2, b1s, b2s, bias)

    tab2 = _sc_scatter(tab, new_rows.reshape(B * Ns, C2),
                       pc_l.reshape(-1), pc_g)

    y = _conv2(tab2.reshape(B, HWp, C2), f1,
               _packtaps(d_w2, r_w2).astype(jnp.bfloat16),
               jnp.concatenate([d_b2, r_b2]).reshape(1, C2))

    out_d = y[..., :C].reshape(B, H, W, C).transpose(0, 3, 1, 2)
    out_r = y[..., C:].reshape(B, H, W, C).transpose(0, 3, 1, 2)
    return out_d, out_r


# speedup vs baseline: 1.1154x; 1.1154x over previous
import functools

import jax
import jax.numpy as jnp
from jax import lax
from jax.experimental import pallas as pl
from jax.experimental.pallas import tpu as pltpu
from jax.experimental.pallas import tpu_sc as plsc

B, Cin, C, H, W = 2, 64, 64, 224, 224
C2 = 2 * C
Ns, K = 12544, 9
HW = H * W
HID = (2 * C + 3) // 2
HWp = HW + 16
NW = 32
NKN = B * K * Ns
NN_PER_W = NKN // NW
GCH = 392
PT_PER_W = (B * Ns) // NW
PT_PER_T = Ns // 16

_MESH = plsc.VectorSubcoreMesh(core_axis_name="c", subcore_axis_name="s")


RB = 1792
NB = HW // RB


def _conv1_body(xt_ref, xm_ref, xb_ref, wtab_ref, wf1_ref, b0_ref, b1_ref,
                tab_ref, f1_ref, cat_ref):
    i = pl.program_id(1)
    zero = jnp.zeros((RB, C2), jnp.float32)
    cat_ref[pl.ds(0, RB), :] = jnp.where(i == 0, zero, xt_ref[0])
    cat_ref[pl.ds(RB, RB), :] = xm_ref[0]
    cat_ref[pl.ds(2 * RB, RB), :] = jnp.where(i == NB - 1, zero, xb_ref[0])
    wiota = lax.broadcasted_iota(jnp.int32, (RB, C2), 0) % W
    m_l = wiota != 0
    m_r = wiota != (W - 1)
    acc0 = jnp.zeros((RB, C2), jnp.float32)
    acc1 = jnp.zeros((RB, C2), jnp.float32)
    for t in range(9):
        dy, dx = t // 3 - 1, t % 3 - 1
        xs = cat_ref[pl.ds(RB + dy * W + dx, RB), :]
        if dx == -1:
            xs = jnp.where(m_l, xs, 0.0)
        elif dx == 1:
            xs = jnp.where(m_r, xs, 0.0)
        acc0 = acc0 + jnp.dot(xs, wtab_ref[t], preferred_element_type=jnp.float32)
        acc1 = acc1 + jnp.dot(xs, wf1_ref[t], preferred_element_type=jnp.float32)
    tab_ref[0] = jnp.maximum(acc0 + b0_ref[...], 0.0)
    f1_ref[0] = acc1 + b1_ref[...]


def _conv1(x_cat, wtab, wf1, b0p, b1p):
    blk = pl.BlockSpec((1, RB, C2), lambda b, i: (b, i, 0))
    full3 = pl.BlockSpec(None, lambda b, i: (0, 0, 0))
    full2 = pl.BlockSpec(None, lambda b, i: (0, 0))
    return pl.pallas_call(
        _conv1_body,
        grid=(B, NB),
        in_specs=[
            pl.BlockSpec((1, RB, C2), lambda b, i: (b, jnp.maximum(i - 1, 0), 0)),
            blk,
            pl.BlockSpec((1, RB, C2),
                         lambda b, i: (b, jnp.minimum(i + 1, NB - 1), 0)),
            full3, full3, full2, full2,
        ],
        out_specs=[
            pl.BlockSpec((1, RB, C2), lambda b, i: (b, i, 0)),
            pl.BlockSpec((1, RB, C2), lambda b, i: (b, i, 0)),
        ],
        out_shape=[
            jax.ShapeDtypeStruct((B, HWp, C2), jnp.float32),
            jax.ShapeDtypeStruct((B, HW, C2), jnp.float32),
        ],
        scratch_shapes=[pltpu.VMEM((3 * RB, C2), jnp.float32)],
    )(x_cat, x_cat, x_cat, wtab, wf1, b0p, b1p)


def _conv2_body(tt_ref, tm_ref, tb_ref, f1_ref, w2_ref, b2_ref,
                out_ref, cat_ref):
    i = pl.program_id(1)
    zero = jnp.zeros((RB, C2), jnp.float32)
    cat_ref[pl.ds(0, RB), :] = jnp.where(i == 0, zero, tt_ref[0])
    cat_ref[pl.ds(RB, RB), :] = tm_ref[0]
    cat_ref[pl.ds(2 * RB, RB), :] = jnp.where(i == NB - 1, zero, tb_ref[0])
    wiota = lax.broadcasted_iota(jnp.int32, (RB, C2), 0) % W
    m_l = wiota != 0
    m_r = wiota != (W - 1)
    acc = jnp.zeros((RB, C2), jnp.float32)
    for t in range(9):
        dy, dx = t // 3 - 1, t % 3 - 1
        xs = cat_ref[pl.ds(RB + dy * W + dx, RB), :]
        if dx == -1:
            xs = jnp.where(m_l, xs, 0.0)
        elif dx == 1:
            xs = jnp.where(m_r, xs, 0.0)
        acc = acc + jnp.dot(xs, w2_ref[t], preferred_element_type=jnp.float32)
    out_ref[0] = jnp.maximum(acc + b2_ref[...] + f1_ref[0], 0.0)


def _conv2(tab2, f1, w2p, b2p):
    full3 = pl.BlockSpec(None, lambda b, i: (0, 0, 0))
    full2 = pl.BlockSpec(None, lambda b, i: (0, 0))
    return pl.pallas_call(
        _conv2_body,
        grid=(B, NB),
        in_specs=[
            pl.BlockSpec((1, RB, C2), lambda b, i: (b, jnp.maximum(i - 1, 0), 0)),
            pl.BlockSpec((1, RB, C2), lambda b, i: (b, i, 0)),
            pl.BlockSpec((1, RB, C2),
                         lambda b, i: (b, jnp.minimum(i + 1, NB - 1), 0)),
            pl.BlockSpec((1, RB, C2), lambda b, i: (b, i, 0)),
            full3, full2,
        ],
        out_specs=pl.BlockSpec((1, RB, C2), lambda b, i: (b, i, 0)),
        out_shape=jax.ShapeDtypeStruct((B, HW, C2), jnp.float32),
        scratch_shapes=[pltpu.VMEM((3 * RB, C2), jnp.float32)],
    )(tab2, tab2, tab2, f1, w2p, b2p)


@functools.partial(
    pl.kernel,
    out_type=(
        jax.ShapeDtypeStruct((NKN, C2), jnp.float32),
        jax.ShapeDtypeStruct((B * Ns, C2), jnp.float32),
    ),
    mesh=_MESH,
    scratch_types=[
        pltpu.VMEM((GCH,), jnp.int32),
        pltpu.VMEM((GCH,), jnp.int32),
        pltpu.VMEM((GCH, C2), jnp.float32),
        pltpu.VMEM((GCH, C2), jnp.float32),
        pltpu.SemaphoreType.DMA,
        pltpu.SemaphoreType.DMA,
    ],
    compiler_params=pltpu.CompilerParams(needs_layout_passes=False),
)
def _sc_gather(tab, nbrs_g, pc_g, nn_out, pt_out, idx_a, idx_b,
               rows_a, rows_b, gsem, osem):
    wid = lax.axis_index("c") * 16 + lax.axis_index("s")
    nb_base = wid * NN_PER_W
    pt_base = wid * PT_PER_W
    nnc = NN_PER_W // GCH
    ptc = PT_PER_W // GCH

    def src_dst(i):
        if i < nnc:
            return nbrs_g, nn_out, nb_base + i * GCH
        j = i - nnc
        return pc_g, pt_out, pt_base + j * GCH

    n = nnc + ptc
    idx_bufs = (idx_a, idx_b)
    row_bufs = (rows_a, rows_b)
    gathers = {}
    outs = {}
    src0, _, base0 = src_dst(0)
    pltpu.sync_copy(src0.at[pl.ds(base0, GCH)], idx_a)
    gathers[0] = pltpu.async_copy(tab.at[idx_a], rows_a, gsem)
    for i in range(n):
        s, nx = i % 2, (i + 1) % 2
        if i + 1 < n:
            srcn, _, basen = src_dst(i + 1)
            pltpu.sync_copy(srcn.at[pl.ds(basen, GCH)], idx_bufs[nx])
        gathers[i].wait()
        if i >= 1:
            outs[i - 1].wait()
        _, dsti, basei = src_dst(i)
        outs[i] = pltpu.async_copy(row_bufs[s], dsti.at[pl.ds(basei, GCH)],
                                   osem)
        if i + 1 < n:
            gathers[i + 1] = pltpu.async_copy(tab.at[idx_bufs[nx]],
                                              row_bufs[nx], gsem)
    outs[n - 1].wait()


BN = 896


def _attn_body(nn_ref, pt_ref, disp_ref, g_ref, pw_ref, wp_ref, w2_ref,
               b1_ref, b2_ref, bias_ref, out_ref):
    pt = pt_ref[0]
    point = (jnp.dot(pt, pw_ref[...], preferred_element_type=jnp.float32)
             + b1_ref[...])
    den = jnp.zeros((BN, 2), jnp.float32)
    accd = jnp.zeros((BN, C), jnp.float32)
    accr = jnp.zeros((BN, C), jnp.float32)
    for k in range(K):
        nk128 = nn_ref[0, k]
        h = (jnp.dot(nk128, g_ref[...], preferred_element_type=jnp.float32)
             + lax.dot_general(disp_ref[0, pl.ds(3 * k, 3)], wp_ref[...],
                               (((0,), (0,)), ((), ())),
                               preferred_element_type=jnp.float32)
             + point)
        h = jnp.where(h >= 0, h, 0.2 * h)
        e = jnp.exp(jnp.dot(h, w2_ref[...], preferred_element_type=jnp.float32)
                    + b2_ref[...])
        den = den + e
        nk = nk128[:, 0:C]
        accd = accd + e[:, 0:1] * nk
        accr = accr + e[:, 1:2] * nk
    inv = 1.0 / den
    accd = accd * inv[:, 0:1]
    accr = accr * inv[:, 1:2]
    acc = jnp.concatenate([accd, accr], axis=1) + bias_ref[...]
    ci = lax.broadcasted_iota(jnp.int32, (BN, C2), 1)
    keep = jnp.logical_and(ci != 0, ci != C)
    out_ref[0] = acc + jnp.where(keep, pt, 0.0)


def _tc_attn(nn, pt, disp, g, pw, wp, w2, b1s, b2s, bias):
    nblk = Ns // BN
    w2d = pl.BlockSpec(None, lambda b, i: (0, 0))
    return pl.pallas_call(
        _attn_body,
        grid=(B, nblk),
        in_specs=[
            pl.BlockSpec((1, K, BN, C2), lambda b, i: (b, 0, i, 0)),
            pl.BlockSpec((1, BN, C2), lambda b, i: (b, i, 0)),
            pl.BlockSpec((1, 3 * K, BN), lambda b, i: (b, 0, i)),
            w2d, w2d, w2d, w2d, w2d, w2d, w2d,
        ],
        out_specs=pl.BlockSpec((1, BN, C2), lambda b, i: (b, i, 0)),
        out_shape=jax.ShapeDtypeStruct((B, Ns, C2), jnp.float32),
    )(nn, pt, disp, g, pw, wp, w2, b1s, b2s, bias)


CPR = HW // 16
CCH = 384
_PASSES = ((0, 384), (384, 384), (768, 16))


@functools.partial(
    pl.kernel,
    out_type=jax.ShapeDtypeStruct((B * HWp, C2), jnp.float32),
    mesh=_MESH,
    scratch_types=[
        pltpu.VMEM((HW,), jnp.int32),
        pltpu.VMEM((Ns,), jnp.int32),
        pltpu.VMEM((PT_PER_T,), jnp.int32),
        pltpu.VMEM((384,), jnp.int32),
        pltpu.VMEM((384, C2), jnp.float32),
        pltpu.SemaphoreType.DMA,
    ],
    compiler_params=pltpu.CompilerParams(needs_layout_passes=False),
)
def _sc_scatter(tab, rows, pc_lf, pc_g, out,
                wm_v, pc_v, msk_v, idx_v, rows_v, sem):
    b = lax.axis_index("c")
    t = lax.axis_index("s")

    row0 = b * HWp + t * CPR

    def cp_chunk(j, _):
        pltpu.sync_copy(tab.at[pl.ds(row0 + j * CCH, CCH)], rows_v)
        pltpu.sync_copy(rows_v, out.at[pl.ds(row0 + j * CCH, CCH)])
        return _

    lax.fori_loop(0, CPR // CCH, cp_chunk, 0)
    tail = CPR - (CPR // CCH) * CCH
    pltpu.sync_copy(tab.at[pl.ds(row0 + CPR - tail, tail)],
                    rows_v.at[pl.ds(0, tail)])
    pltpu.sync_copy(rows_v.at[pl.ds(0, tail)],
                    out.at[pl.ds(row0 + CPR - tail, tail)])

    pltpu.sync_copy(pc_lf.at[pl.ds(b * Ns, Ns)], pc_v)
    lanes = lax.iota(jnp.int32, 16)

    def scat(i, _):
        idx16 = pc_v[pl.ds(i * 16, 16)]
        plsc.store_scatter(wm_v, [idx16], lanes + i * 16)
        return _

    lax.fori_loop(0, Ns // 16, scat, 0)

    tb = t * PT_PER_T

    def wmask(i, _):
        idx16 = pc_v[pl.ds(tb + i * 16, 16)]
        got = plsc.load_gather(wm_v, [idx16])
        msk_v[pl.ds(i * 16, 16)] = jnp.where(got == lanes + (tb + i * 16), 1, 0)
        return _

    lax.fori_loop(0, PT_PER_T // 16, wmask, 0)

    plsc.subcore_barrier()

    sentinel = b * HWp + HW + t
    gstart = b * Ns + tb

    for off, npt in _PASSES:
        pltpu.sync_copy(pc_g.at[pl.ds(gstart + off, npt)],
                        idx_v.at[pl.ds(0, npt)])
        pltpu.sync_copy(rows.at[pl.ds(gstart + off, npt)],
                        rows_v.at[pl.ds(0, npt)])

        def effidx(i, _):
            w16 = msk_v[pl.ds(off + i * 16, 16)]
            i16 = idx_v[pl.ds(i * 16, 16)]
            idx_v[pl.ds(i * 16, 16)] = jnp.where(w16 == 1, i16, sentinel)
            return _

        lax.fori_loop(0, npt // 16, effidx, 0)

        pltpu.async_copy(rows_v.at[pl.ds(0, npt)],
                         out.at[idx_v.at[pl.ds(0, npt)]], sem).wait()


def _taps(w):
    return w.transpose(2, 3, 1, 0).reshape(9, Cin, C)


def _packtaps(wd, wr):
    z = jnp.zeros((9, C2, C2), jnp.float32)
    return z.at[:, :C, :C].set(_taps(wd)).at[:, C:, C:].set(_taps(wr))


def kernel(rgb, sdepth, pc_idx, nbrs_idx, nbrs_disp,
           d_w0, d_b0, d_w1, d_b1, d_w2, d_b2,
           r_w0, r_b0, r_w1, r_b1, r_w2, r_b2,
           d_mlp_w1, d_mlp_b1, d_mlp_w2, d_mlp_b2,
           r_mlp_w1, r_mlp_b1, r_mlp_w2, r_mlp_b2,
           d_bias, r_bias):
    x_cat = jnp.concatenate([sdepth, rgb], axis=1).transpose(0, 2, 3, 1)
    x_cat = x_cat.reshape(B, HW, C2)

    tab3, f1 = _conv1(x_cat,
                      _packtaps(d_w0, r_w0),
                      _packtaps(d_w1, r_w1),
                      jnp.concatenate([d_b0, r_b0]).reshape(1, C2),
                      jnp.concatenate([d_b1, r_b1]).reshape(1, C2))
    tab = tab3.reshape(B * HWp, C2)

    off_b = jnp.arange(B, dtype=jnp.int32) * HWp
    pc_l = pc_idx.reshape(B, Ns).astype(jnp.int32)
    pc_g = (pc_l + off_b[:, None]).reshape(-1)
    nbrs = nbrs_idx.reshape(B, Ns, K).astype(jnp.int32).transpose(0, 2, 1)
    nbrs_g = (nbrs + off_b[:, None, None]).reshape(-1)
    disp_t = nbrs_disp.transpose(0, 3, 1, 2).reshape(B, 3 * K, Ns)

    nn_f, pt_f = _sc_gather(tab, nbrs_g, pc_g)

    a1d, a2d, a3d = (d_mlp_w1[:, :C], d_mlp_w1[:, C:2 * C], d_mlp_w1[:, 2 * C:])
    a1r, a2r, a3r = (r_mlp_w1[:, :C], r_mlp_w1[:, C:2 * C], r_mlp_w1[:, 2 * C:])
    g_w = jnp.concatenate([(a1d + a2d).T, (a1r + a2r).T], axis=1)
    g_w = jnp.pad(g_w, ((0, C), (0, 0)))
    pw = jnp.concatenate(
        [jnp.concatenate([-a1d.T, -a1r.T], axis=1),
         jnp.concatenate([-a2d.T, -a2r.T], axis=1)], axis=0)
    wp = jnp.concatenate([a3d.T, a3r.T], axis=1)
    w2 = jnp.zeros((2 * HID, 2), jnp.float32)
    w2 = w2.at[:HID, 0].set(d_mlp_w2[0]).at[HID:, 1].set(r_mlp_w2[0])
    b1s = jnp.concatenate([d_mlp_b1, r_mlp_b1]).reshape(1, 2 * HID)
    b2s = jnp.concatenate([d_mlp_b2, r_mlp_b2]).reshape(1, 2)
    bias = jnp.concatenate([d_bias, r_bias]).reshape(1, C2)

    new_rows = _tc_attn(nn_f.reshape(B, K, Ns, C2), pt_f.reshape(B, Ns, C2),
                        disp_t, g_w, pw, wp, w2, b1s, b2s, bias)

    tab2 = _sc_scatter(tab, new_rows.reshape(B * Ns, C2),
                       pc_l.reshape(-1), pc_g)

    y = _conv2(tab2.reshape(B, HWp, C2), f1,
               _packtaps(d_w2, r_w2),
               jnp.concatenate([d_b2, r_b2]).reshape(1, C2))

    out_d = y[..., :C].reshape(B, H, W, C).transpose(0, 3, 1, 2)
    out_r = y[..., C:].reshape(B, H, W, C).transpose(0, 3, 1, 2)
    return out_d, out_r


# speedup vs baseline: 1.1546x; 1.0352x over previous
import functools

import jax
import jax.numpy as jnp
from jax import lax
from jax.experimental import pallas as pl
from jax.experimental.pallas import tpu as pltpu
from jax.experimental.pallas import tpu_sc as plsc

B, Cin, C, H, W = 2, 64, 64, 224, 224
C2 = 2 * C
Ns, K = 12544, 9
HW = H * W
HID = (2 * C + 3) // 2
HWp = HW + 16
NW = 32
NKN = B * K * Ns
NN_PER_W = NKN // NW
GCH = 392
PT_PER_W = (B * Ns) // NW
PT_PER_T = Ns // 16

_MESH = plsc.VectorSubcoreMesh(core_axis_name="c", subcore_axis_name="s")


RB = 3584
NB = HW // RB


def _conv1_body(xt_ref, xm_ref, xb_ref, wtab_ref, wf1_ref, b0_ref, b1_ref,
                tab_ref, f1_ref, cat_ref):
    i = pl.program_id(1)
    zero = jnp.zeros((RB, C2), jnp.float32)
    cat_ref[pl.ds(0, RB), :] = jnp.where(i == 0, zero, xt_ref[0])
    cat_ref[pl.ds(RB, RB), :] = xm_ref[0]
    cat_ref[pl.ds(2 * RB, RB), :] = jnp.where(i == NB - 1, zero, xb_ref[0])
    wiota = lax.broadcasted_iota(jnp.int32, (RB, C2), 0) % W
    m_l = wiota != 0
    m_r = wiota != (W - 1)
    acc0 = jnp.zeros((RB, C2), jnp.float32)
    acc1 = jnp.zeros((RB, C2), jnp.float32)
    for t in range(9):
        dy, dx = t // 3 - 1, t % 3 - 1
        xs = cat_ref[pl.ds(RB + dy * W + dx, RB), :]
        if dx == -1:
            xs = jnp.where(m_l, xs, 0.0)
        elif dx == 1:
            xs = jnp.where(m_r, xs, 0.0)
        acc0 = acc0 + jnp.dot(xs, wtab_ref[t], preferred_element_type=jnp.float32)
        acc1 = acc1 + jnp.dot(xs, wf1_ref[t], preferred_element_type=jnp.float32)
    tab_ref[0] = jnp.maximum(acc0 + b0_ref[...], 0.0)
    f1_ref[0] = acc1 + b1_ref[...]


def _conv1(x_cat, wtab, wf1, b0p, b1p):
    blk = pl.BlockSpec((1, RB, C2), lambda b, i: (b, i, 0))
    full3 = pl.BlockSpec(None, lambda b, i: (0, 0, 0))
    full2 = pl.BlockSpec(None, lambda b, i: (0, 0))
    return pl.pallas_call(
        _conv1_body,
        grid=(B, NB),
        in_specs=[
            pl.BlockSpec((1, RB, C2), lambda b, i: (b, jnp.maximum(i - 1, 0), 0)),
            blk,
            pl.BlockSpec((1, RB, C2),
                         lambda b, i: (b, jnp.minimum(i + 1, NB - 1), 0)),
            full3, full3, full2, full2,
        ],
        out_specs=[
            pl.BlockSpec((1, RB, C2), lambda b, i: (b, i, 0)),
            pl.BlockSpec((1, RB, C2), lambda b, i: (b, i, 0)),
        ],
        out_shape=[
            jax.ShapeDtypeStruct((B, HWp, C2), jnp.float32),
            jax.ShapeDtypeStruct((B, HW, C2), jnp.float32),
        ],
        scratch_shapes=[pltpu.VMEM((3 * RB, C2), jnp.float32)],
    )(x_cat, x_cat, x_cat, wtab, wf1, b0p, b1p)


def _conv2_body(tt_ref, tm_ref, tb_ref, f1_ref, w2_ref, b2_ref,
                out_ref, cat_ref):
    i = pl.program_id(1)
    zero = jnp.zeros((RB, C2), jnp.float32)
    cat_ref[pl.ds(0, RB), :] = jnp.where(i == 0, zero, tt_ref[0])
    cat_ref[pl.ds(RB, RB), :] = tm_ref[0]
    cat_ref[pl.ds(2 * RB, RB), :] = jnp.where(i == NB - 1, zero, tb_ref[0])
    wiota = lax.broadcasted_iota(jnp.int32, (RB, C2), 0) % W
    m_l = wiota != 0
    m_r = wiota != (W - 1)
    acc = jnp.zeros((RB, C2), jnp.float32)
    for t in range(9):
        dy, dx = t // 3 - 1, t % 3 - 1
        xs = cat_ref[pl.ds(RB + dy * W + dx, RB), :]
        if dx == -1:
            xs = jnp.where(m_l, xs, 0.0)
        elif dx == 1:
            xs = jnp.where(m_r, xs, 0.0)
        acc = acc + jnp.dot(xs, w2_ref[t], preferred_element_type=jnp.float32)
    out_ref[0] = jnp.maximum(acc + b2_ref[...] + f1_ref[0], 0.0)


def _conv2(tab2, f1, w2p, b2p):
    full3 = pl.BlockSpec(None, lambda b, i: (0, 0, 0))
    full2 = pl.BlockSpec(None, lambda b, i: (0, 0))
    return pl.pallas_call(
        _conv2_body,
        grid=(B, NB),
        in_specs=[
            pl.BlockSpec((1, RB, C2), lambda b, i: (b, jnp.maximum(i - 1, 0), 0)),
            pl.BlockSpec((1, RB, C2), lambda b, i: (b, i, 0)),
            pl.BlockSpec((1, RB, C2),
                         lambda b, i: (b, jnp.minimum(i + 1, NB - 1), 0)),
            pl.BlockSpec((1, RB, C2), lambda b, i: (b, i, 0)),
            full3, full2,
        ],
        out_specs=pl.BlockSpec((1, RB, C2), lambda b, i: (b, i, 0)),
        out_shape=jax.ShapeDtypeStruct((B, HW, C2), jnp.float32),
        scratch_shapes=[pltpu.VMEM((3 * RB, C2), jnp.float32)],
    )(tab2, tab2, tab2, f1, w2p, b2p)


@functools.partial(
    pl.kernel,
    out_type=(
        jax.ShapeDtypeStruct((NKN, C2), jnp.float32),
        jax.ShapeDtypeStruct((B * Ns, C2), jnp.float32),
    ),
    mesh=_MESH,
    scratch_types=[
        pltpu.VMEM((GCH,), jnp.int32),
        pltpu.VMEM((GCH,), jnp.int32),
        pltpu.VMEM((GCH, C2), jnp.float32),
        pltpu.VMEM((GCH, C2), jnp.float32),
        pltpu.SemaphoreType.DMA,
        pltpu.SemaphoreType.DMA,
    ],
    compiler_params=pltpu.CompilerParams(needs_layout_passes=False),
)
def _sc_gather(tab, nbrs_g, pc_g, nn_out, pt_out, idx_a, idx_b,
               rows_a, rows_b, gsem, osem):
    wid = lax.axis_index("c") * 16 + lax.axis_index("s")
    nb_base = wid * NN_PER_W
    pt_base = wid * PT_PER_W
    nnc = NN_PER_W // GCH
    ptc = PT_PER_W // GCH

    def src_dst(i):
        if i < nnc:
            return nbrs_g, nn_out, nb_base + i * GCH
        j = i - nnc
        return pc_g, pt_out, pt_base + j * GCH

    n = nnc + ptc
    idx_bufs = (idx_a, idx_b)
    row_bufs = (rows_a, rows_b)
    gathers = {}
    outs = {}
    src0, _, base0 = src_dst(0)
    pltpu.sync_copy(src0.at[pl.ds(base0, GCH)], idx_a)
    gathers[0] = pltpu.async_copy(tab.at[idx_a], rows_a, gsem)
    for i in range(n):
        s, nx = i % 2, (i + 1) % 2
        if i + 1 < n:
            srcn, _, basen = src_dst(i + 1)
            pltpu.sync_copy(srcn.at[pl.ds(basen, GCH)], idx_bufs[nx])
        gathers[i].wait()
        if i >= 1:
            outs[i - 1].wait()
        _, dsti, basei = src_dst(i)
        outs[i] = pltpu.async_copy(row_bufs[s], dsti.at[pl.ds(basei, GCH)],
                                   osem)
        if i + 1 < n:
            gathers[i + 1] = pltpu.async_copy(tab.at[idx_bufs[nx]],
                                              row_bufs[nx], gsem)
    outs[n - 1].wait()


BN = 896


def _attn_body(nn_ref, pt_ref, disp_ref, g_ref, pw_ref, wp_ref, w2_ref,
               b1_ref, b2_ref, bias_ref, out_ref):
    pt = pt_ref[0]
    point = (jnp.dot(pt, pw_ref[...], preferred_element_type=jnp.float32)
             + b1_ref[...])
    den = jnp.zeros((BN, 2), jnp.float32)
    accd = jnp.zeros((BN, C), jnp.float32)
    accr = jnp.zeros((BN, C), jnp.float32)
    for k in range(K):
        nk128 = nn_ref[0, k]
        h = (jnp.dot(nk128, g_ref[...], preferred_element_type=jnp.float32)
             + lax.dot_general(disp_ref[0, pl.ds(3 * k, 3)], wp_ref[...],
                               (((0,), (0,)), ((), ())),
                               preferred_element_type=jnp.float32)
             + point)
        h = jnp.where(h >= 0, h, 0.2 * h)
        e = jnp.exp(jnp.dot(h, w2_ref[...], preferred_element_type=jnp.float32)
                    + b2_ref[...])
        den = den + e
        nk = nk128[:, 0:C]
        accd = accd + e[:, 0:1] * nk
        accr = accr + e[:, 1:2] * nk
    inv = 1.0 / den
    accd = accd * inv[:, 0:1]
    accr = accr * inv[:, 1:2]
    acc = jnp.concatenate([accd, accr], axis=1) + bias_ref[...]
    ci = lax.broadcasted_iota(jnp.int32, (BN, C2), 1)
    keep = jnp.logical_and(ci != 0, ci != C)
    out_ref[0] = acc + jnp.where(keep, pt, 0.0)


def _tc_attn(nn, pt, disp, g, pw, wp, w2, b1s, b2s, bias):
    nblk = Ns // BN
    w2d = pl.BlockSpec(None, lambda b, i: (0, 0))
    return pl.pallas_call(
        _attn_body,
        grid=(B, nblk),
        in_specs=[
            pl.BlockSpec((1, K, BN, C2), lambda b, i: (b, 0, i, 0)),
            pl.BlockSpec((1, BN, C2), lambda b, i: (b, i, 0)),
            pl.BlockSpec((1, 3 * K, BN), lambda b, i: (b, 0, i)),
            w2d, w2d, w2d, w2d, w2d, w2d, w2d,
        ],
        out_specs=pl.BlockSpec((1, BN, C2), lambda b, i: (b, i, 0)),
        out_shape=jax.ShapeDtypeStruct((B, Ns, C2), jnp.float32),
    )(nn, pt, disp, g, pw, wp, w2, b1s, b2s, bias)


CPR = HW // 16
CCH = 112
_PASSES = ((0, 384), (384, 384), (768, 16))


@functools.partial(
    pl.kernel,
    out_type=jax.ShapeDtypeStruct((B * HWp, C2), jnp.float32),
    mesh=_MESH,
    scratch_types=[
        pltpu.VMEM((HW,), jnp.int32),
        pltpu.VMEM((Ns,), jnp.int32),
        pltpu.VMEM((PT_PER_T,), jnp.int32),
        pltpu.VMEM((384,), jnp.int32),
        pltpu.VMEM((384, C2), jnp.float32),
        pltpu.VMEM((CCH, C2), jnp.float32),
        pltpu.SemaphoreType.DMA,
        pltpu.SemaphoreType.DMA,
    ],
    compiler_params=pltpu.CompilerParams(needs_layout_passes=False),
)
def _sc_scatter(tab, rows, pc_lf, pc_g, out,
                wm_v, pc_v, msk_v, idx_v, rows_v, cp_b, isem, sem):
    b = lax.axis_index("c")
    t = lax.axis_index("s")

    row0 = b * HWp + t * CPR
    ncp = CPR // CCH
    bufs = (rows_v.at[pl.ds(0, CCH)], cp_b)
    ins = {}
    outs = {}
    ins[0] = pltpu.async_copy(tab.at[pl.ds(row0, CCH)], bufs[0], isem)
    for j in range(ncp):
        s, nx = j % 2, (j + 1) % 2
        if j + 1 < ncp:
            if j >= 1:
                outs[j - 1].wait()
            ins[j + 1] = pltpu.async_copy(
                tab.at[pl.ds(row0 + (j + 1) * CCH, CCH)], bufs[nx], isem)
        ins[j].wait()
        outs[j] = pltpu.async_copy(bufs[s],
                                   out.at[pl.ds(row0 + j * CCH, CCH)], sem)
    outs[ncp - 1].wait()

    pltpu.sync_copy(pc_lf.at[pl.ds(b * Ns, Ns)], pc_v)
    lanes = lax.iota(jnp.int32, 16)

    def scat(i, _):
        idx16 = pc_v[pl.ds(i * 16, 16)]
        plsc.store_scatter(wm_v, [idx16], lanes + i * 16)
        return _

    lax.fori_loop(0, Ns // 16, scat, 0)

    tb = t * PT_PER_T

    def wmask(i, _):
        idx16 = pc_v[pl.ds(tb + i * 16, 16)]
        got = plsc.load_gather(wm_v, [idx16])
        msk_v[pl.ds(i * 16, 16)] = jnp.where(got == lanes + (tb + i * 16), 1, 0)
        return _

    lax.fori_loop(0, PT_PER_T // 16, wmask, 0)

    plsc.subcore_barrier()

    sentinel = b * HWp + HW + t
    gstart = b * Ns + tb

    for off, npt in _PASSES:
        pltpu.sync_copy(pc_g.at[pl.ds(gstart + off, npt)],
                        idx_v.at[pl.ds(0, npt)])
        pltpu.sync_copy(rows.at[pl.ds(gstart + off, npt)],
                        rows_v.at[pl.ds(0, npt)])

        def effidx(i, _):
            w16 = msk_v[pl.ds(off + i * 16, 16)]
            i16 = idx_v[pl.ds(i * 16, 16)]
            idx_v[pl.ds(i * 16, 16)] = jnp.where(w16 == 1, i16, sentinel)
            return _

        lax.fori_loop(0, npt // 16, effidx, 0)

        pltpu.async_copy(rows_v.at[pl.ds(0, npt)],
                         out.at[idx_v.at[pl.ds(0, npt)]], sem).wait()


def _taps(w):
    return w.transpose(2, 3, 1, 0).reshape(9, Cin, C)


def _packtaps(wd, wr):
    z = jnp.zeros((9, C2, C2), jnp.float32)
    return z.at[:, :C, :C].set(_taps(wd)).at[:, C:, C:].set(_taps(wr))


def kernel(rgb, sdepth, pc_idx, nbrs_idx, nbrs_disp,
           d_w0, d_b0, d_w1, d_b1, d_w2, d_b2,
           r_w0, r_b0, r_w1, r_b1, r_w2, r_b2,
           d_mlp_w1, d_mlp_b1, d_mlp_w2, d_mlp_b2,
           r_mlp_w1, r_mlp_b1, r_mlp_w2, r_mlp_b2,
           d_bias, r_bias):
    x_cat = jnp.concatenate([sdepth, rgb], axis=1).transpose(0, 2, 3, 1)
    x_cat = x_cat.reshape(B, HW, C2)

    tab3, f1 = _conv1(x_cat,
                      _packtaps(d_w0, r_w0),
                      _packtaps(d_w1, r_w1),
                      jnp.concatenate([d_b0, r_b0]).reshape(1, C2),
                      jnp.concatenate([d_b1, r_b1]).reshape(1, C2))
    tab = tab3.reshape(B * HWp, C2)

    off_b = jnp.arange(B, dtype=jnp.int32) * HWp
    pc_l = pc_idx.reshape(B, Ns).astype(jnp.int32)
    pc_g = (pc_l + off_b[:, None]).reshape(-1)
    nbrs = nbrs_idx.reshape(B, Ns, K).astype(jnp.int32).transpose(0, 2, 1)
    nbrs_g = (nbrs + off_b[:, None, None]).reshape(-1)
    disp_t = nbrs_disp.transpose(0, 3, 1, 2).reshape(B, 3 * K, Ns)

    nn_f, pt_f = _sc_gather(tab, nbrs_g, pc_g)

    a1d, a2d, a3d = (d_mlp_w1[:, :C], d_mlp_w1[:, C:2 * C], d_mlp_w1[:, 2 * C:])
    a1r, a2r, a3r = (r_mlp_w1[:, :C], r_mlp_w1[:, C:2 * C], r_mlp_w1[:, 2 * C:])
    g_w = jnp.concatenate([(a1d + a2d).T, (a1r + a2r).T], axis=1)
    g_w = jnp.pad(g_w, ((0, C), (0, 0)))
    pw = jnp.concatenate(
        [jnp.concatenate([-a1d.T, -a1r.T], axis=1),
         jnp.concatenate([-a2d.T, -a2r.T], axis=1)], axis=0)
    wp = jnp.concatenate([a3d.T, a3r.T], axis=1)
    w2 = jnp.zeros((2 * HID, 2), jnp.float32)
    w2 = w2.at[:HID, 0].set(d_mlp_w2[0]).at[HID:, 1].set(r_mlp_w2[0])
    b1s = jnp.concatenate([d_mlp_b1, r_mlp_b1]).reshape(1, 2 * HID)
    b2s = jnp.concatenate([d_mlp_b2, r_mlp_b2]).reshape(1, 2)
    bias = jnp.concatenate([d_bias, r_bias]).reshape(1, C2)

    new_rows = _tc_attn(nn_f.reshape(B, K, Ns, C2), pt_f.reshape(B, Ns, C2),
                        disp_t, g_w, pw, wp, w2, b1s, b2s, bias)

    tab2 = _sc_scatter(tab, new_rows.reshape(B * Ns, C2),
                       pc_l.reshape(-1), pc_g)

    y = _conv2(tab2.reshape(B, HWp, C2), f1,
               _packtaps(d_w2, r_w2),
               jnp.concatenate([d_b2, r_b2]).reshape(1, C2))

    out_d = y[..., :C].reshape(B, H, W, C).transpose(0, 3, 1, 2)
    out_r = y[..., C:].reshape(B, H, W, C).transpose(0, 3, 1, 2)
    return out_d, out_r
